# baseline (speedup 1.0000x reference)
import functools

import numpy as np
import jax
import jax.numpy as jnp
from jax.experimental import pallas as pl
from jax.experimental.pallas import tpu as pltpu


def _up_np(mode, k):
    m = np.zeros((2 * k, k), np.float32)
    i = np.arange(k)
    if mode == "n":
        m[2 * i, i] = 1.0
        m[2 * i + 1, i] = 1.0
    else:
        np.add.at(m, (2 * i, np.maximum(i - 1, 0)), 0.25)
        np.add.at(m, (2 * i, i), 0.75)
        np.add.at(m, (2 * i + 1, i), 0.75)
        np.add.at(m, (2 * i + 1, np.minimum(i + 1, k - 1)), 0.25)
    return m


def _band(w3, h):
    eyes = [jnp.asarray(np.eye(h, k=1 - kx, dtype=np.float32))
            for kx in range(3)]
    return [sum(jnp.kron(w3[ky, kx], eyes[kx]) for kx in range(3))
            for ky in range(3)]


def _vsh(x, d, h):
    z = jnp.zeros((1, x.shape[1]), x.dtype)
    t = (jnp.concatenate([z, x[:-1]], 0) if d < 0
         else jnp.concatenate([x[1:], z], 0))
    if x.shape[0] == h:
        return t
    idx = jax.lax.broadcasted_iota(jnp.int32, x.shape, 0)
    keep = (idx % h) != (0 if d < 0 else h - 1)
    return jnp.where(keep, t, jnp.zeros_like(t))


def _fb_kernel(x0_ref, x1_ref, x2_ref, x3_ref, x4_ref, ul_ref, c32_ref,
               cf_ref, bcat_ref, bias_ref, urb_ref, o_ref, *, plan, catb,
               ulb_off, base, nb):
    x_refs = (x0_ref, x1_ref, x2_ref, x3_ref, x4_ref)
    f32 = jnp.float32

    def dot(a, b, prec=None):
        return jnp.dot(a, b, preferred_element_type=f32, precision=prec)

    hi = jax.lax.Precision.HIGHEST

    def conv_step(X, st):
        if st["ul"] is not None:
            uo, h = st["ul"]
            X = dot(ul_ref[uo:uo + nb * 2 * h, 0:nb * h], X,
                    None if st["final"] else hi)
        h2 = st["hout"]
        cref = cf_ref if st["final"] else c32_ref
        if st["final"]:
            X = X.astype(jnp.bfloat16)
        co_, rows, cols = st["c"]
        acc = bias_ref[st["b"]:st["b"] + 1,
                       0:(8 * h2 if st["final"] else 4 * h2)]
        parts = []
        for ky in range(3):
            T = X if ky == 1 else _vsh(X, ky - 1, h2)
            C = cref[co_ + ky * rows:co_ + (ky + 1) * rows, 0:cols]
            parts.append(dot(T, C, None if st["final"] else hi))
        return jnp.maximum(acc + (parts[0] + parts[1]) + parts[2], 0.0)

    Xs = [jnp.concatenate(
        [jnp.concatenate([x_refs[i][b, c] for c in range(4)], axis=1)
         for b in range(nb)], axis=0) for i in range(5)]
    kc = 40 * base
    kp = 8 * base
    parts = []
    for j in range(5):
        for i in range(5):
            if j < len(plan[i]):
                Xs[i] = conv_step(Xs[i], plan[i][j])
                if j == len(plan[i]) - 1:
                    Xb = Xs[i].astype(jnp.bfloat16)
                    p = None
                    for ky in range(3):
                        T = Xb if ky == 1 else _vsh(Xb, ky - 1, base)
                        B = bcat_ref[ky * kc + i * kp:ky * kc + (i + 1) * kp,
                                     :]
                        d = dot(T, B)
                        p = d if p is None else p + d
                    parts.append(p)

    acc = bias_ref[catb:catb + 1, 0:8 * base]
    while len(parts) > 1:
        parts = [sum(parts[k:k + 2]) for k in range(0, len(parts), 2)]
    acc = acc + parts[0]
    ulb = ul_ref[ulb_off:ulb_off + nb * 2 * base, 0:nb * base]
    Z = dot(ulb, acc)
    for co in range(8):
        O = dot(Z[:, co * base:(co + 1) * base], urb_ref[...])
        for b in range(nb):
            o_ref[b, co] = O[b * 2 * base:(b + 1) * 2 * base, :]


def kernel(x0, x1, x2, x3, x4, w_0_0, b_0_0, w_0_1, b_0_1, w_0_2, b_0_2,
           w_0_3, b_0_3, w_0_4, b_0_4, w_1_0, b_1_0, w_1_1, b_1_1, w_1_2,
           b_1_2, w_1_3, b_1_3, w_2_0, b_2_0, w_2_1, b_2_1, w_2_2, b_2_2,
           w_3_0, b_3_0, w_3_1, b_3_1, w_4_0, b_4_0, conv_w, conv_b):
    ws = [[(w_0_0, b_0_0), (w_0_1, b_0_1), (w_0_2, b_0_2), (w_0_3, b_0_3),
           (w_0_4, b_0_4)],
          [(w_1_0, b_1_0), (w_1_1, b_1_1), (w_1_2, b_1_2), (w_1_3, b_1_3)],
          [(w_2_0, b_2_0), (w_2_1, b_2_1), (w_2_2, b_2_2)],
          [(w_3_0, b_3_0), (w_3_1, b_3_1)],
          [(w_4_0, b_4_0)]]
    xs = [x0, x1, x2, x3, x4]
    n = x0.shape[0]
    nb = 2 if n % 2 == 0 else 1
    base = x4.shape[-1]
    sizes = [x.shape[-1] for x in xs]
    eye_b = np.eye(nb, dtype=np.float32)

    ul_blocks, ul_offs = [], {}
    row = 0
    need_ul = set()
    for i in range(5):
        h = sizes[i]
        for j in range(1, 5 - i):
            need_ul.add(("b" if j == 3 else "n", h))
            h *= 2
    ul_w = nb * base
    for key in sorted(need_ul):
        m = np.kron(eye_b, _up_np(key[0], key[1]))
        ul_blocks.append(np.pad(m, ((0, 0), (0, ul_w - m.shape[1]))))
        ul_offs[key] = row
        row += m.shape[0]
    ulb_np = _up_np("b", base)
    ulb_off = row
    m = np.kron(eye_b, ulb_np)
    ul_blocks.append(np.pad(m, ((0, 0), (0, ul_w - m.shape[1]))))
    ulpack = jnp.asarray(np.concatenate(ul_blocks, 0))
    urbil = jnp.asarray(np.ascontiguousarray(ulb_np.T))

    c32_rows, cf_rows, bias_rows = [], [], []
    c32_off = cf_off = 0
    plan = []
    for i in range(5):
        h = sizes[i]
        steps = []
        for j in range(5 - i):
            w3, b = ws[i][j]
            final = j == 4 - i
            if j == 0:
                ul = None
                h2 = h
                bands = _band(w3, h2)
            else:
                mode = "b" if j == 3 else "n"
                ul = (ul_offs[(mode, h)], h)
                h2 = 2 * h
                rblk = jnp.asarray(np.kron(np.eye(4, dtype=np.float32),
                                           _up_np(mode, h).T))
                bands = [jnp.dot(rblk, m) for m in _band(w3, h2)]
            rows, cols = bands[0].shape
            if final:
                off = cf_off
                cf_rows += [jnp.pad(m, ((0, 0), (0, 1024 - cols)))
                            for m in bands]
                cf_off += 3 * rows
            else:
                off = c32_off
                c32_rows += [jnp.pad(m, ((0, 0), (0, 2 * base - cols)))
                             for m in bands]
                c32_off += 3 * rows
            brow = jnp.pad(jnp.repeat(b, h2), (0, 1024 - b.shape[0] * h2))
            steps.append({"ul": ul, "c": (off, rows, cols), "final": final,
                          "hout": h2, "b": len(bias_rows)})
            bias_rows.append(brow)
            h = h2
        plan.append(steps)

    bcat = [sum(jnp.kron(conv_w[ky, kx],
                         jnp.asarray(np.eye(base, k=1 - kx,
                                            dtype=np.float32)))
                for kx in range(3)) for ky in range(3)]
    bcat = jnp.concatenate(bcat, 0).astype(jnp.bfloat16)
    catb = len(bias_rows)
    bias_rows.append(jnp.pad(jnp.repeat(conv_b, base), (0, 1024 - 8 * base)))

    c32 = jnp.concatenate(c32_rows, 0)
    cf = jnp.concatenate(cf_rows, 0).astype(jnp.bfloat16)
    biasp = jnp.stack(bias_rows, 0)

    kfn = functools.partial(_fb_kernel, plan=plan, catb=catb,
                            ulb_off=ulb_off, base=base, nb=nb)
    mats = [ulpack, c32, cf, bcat, biasp, urbil]

    def whole(a):
        return pl.BlockSpec(a.shape, lambda bi: (0,) * a.ndim)

    return pl.pallas_call(
        kfn,
        out_shape=jax.ShapeDtypeStruct((n, 8, 2 * base, 2 * base),
                                       jnp.float32),
        grid=(n // nb,),
        in_specs=([pl.BlockSpec((nb, 4, s, s), lambda bi: (bi, 0, 0, 0))
                   for s in sizes] + [whole(a) for a in mats]),
        out_specs=pl.BlockSpec((nb, 8, 2 * base, 2 * base),
                               lambda bi: (bi, 0, 0, 0)),
        compiler_params=pltpu.CompilerParams(
            dimension_semantics=("parallel",),
            vmem_limit_bytes=60 * 1024 * 1024),
    )(*xs, *mats)


# speedup vs baseline: 1.2312x; 1.2312x over previous
import functools

import numpy as np
import jax
import jax.numpy as jnp
from jax.experimental import pallas as pl
from jax.experimental.pallas import tpu as pltpu


def _up_np(mode, k):
    m = np.zeros((2 * k, k), np.float32)
    i = np.arange(k)
    if mode == "n":
        m[2 * i, i] = 1.0
        m[2 * i + 1, i] = 1.0
    else:
        np.add.at(m, (2 * i, np.maximum(i - 1, 0)), 0.25)
        np.add.at(m, (2 * i, i), 0.75)
        np.add.at(m, (2 * i + 1, i), 0.75)
        np.add.at(m, (2 * i + 1, np.minimum(i + 1, k - 1)), 0.25)
    return m


def _band(w3, h):
    eyes = [jnp.asarray(np.eye(h, k=1 - kx, dtype=np.float32))
            for kx in range(3)]
    return [sum(jnp.kron(w3[ky, kx], eyes[kx]) for kx in range(3))
            for ky in range(3)]


def _vsh(x, d, h):
    z = jnp.zeros((1, x.shape[1]), x.dtype)
    t = (jnp.concatenate([z, x[:-1]], 0) if d < 0
         else jnp.concatenate([x[1:], z], 0))
    if x.shape[0] == h:
        return t
    idx = jax.lax.broadcasted_iota(jnp.int32, x.shape, 0)
    keep = (idx % h) != (0 if d < 0 else h - 1)
    return jnp.where(keep, t, jnp.zeros_like(t))


def _fb_kernel(x0_ref, x1_ref, x2_ref, x3_ref, x4_ref, ul_ref, c32_ref,
               cf_ref, bcat_ref, bias_ref, urb_ref, o_ref, *, plan, catb,
               ulb_off, base, nb):
    x_refs = (x0_ref, x1_ref, x2_ref, x3_ref, x4_ref)
    f32 = jnp.float32

    def dot(a, b):
        return jnp.dot(a, b, preferred_element_type=f32)

    def conv_step(X, st):
        if st["ul"] is not None:
            uo, h = st["ul"]
            X = dot(ul_ref[uo:uo + nb * 2 * h, 0:nb * h], X)
        h2 = st["hout"]
        cref = cf_ref if st["final"] else c32_ref
        if st["final"]:
            X = X.astype(jnp.bfloat16)
        co_, rows, cols = st["c"]
        acc = bias_ref[st["b"]:st["b"] + 1,
                       0:(8 * h2 if st["final"] else 4 * h2)]
        parts = []
        for ky in range(3):
            T = X if ky == 1 else _vsh(X, ky - 1, h2)
            C = cref[co_ + ky * rows:co_ + (ky + 1) * rows, 0:cols]
            parts.append(dot(T, C))
        return jnp.maximum(acc + (parts[0] + parts[1]) + parts[2], 0.0)

    Xs = [jnp.concatenate(
        [jnp.concatenate([x_refs[i][b, c] for c in range(4)], axis=1)
         for b in range(nb)], axis=0) for i in range(5)]
    kc = 40 * base
    kp = 8 * base
    parts = []
    for j in range(5):
        for i in range(5):
            if j < len(plan[i]):
                Xs[i] = conv_step(Xs[i], plan[i][j])
                if j == len(plan[i]) - 1:
                    Xb = Xs[i].astype(jnp.bfloat16)
                    p = None
                    for ky in range(3):
                        T = Xb if ky == 1 else _vsh(Xb, ky - 1, base)
                        B = bcat_ref[ky * kc + i * kp:ky * kc + (i + 1) * kp,
                                     :]
                        d = dot(T, B)
                        p = d if p is None else p + d
                    parts.append(p)

    acc = bias_ref[catb:catb + 1, 0:8 * base]
    while len(parts) > 1:
        parts = [sum(parts[k:k + 2]) for k in range(0, len(parts), 2)]
    acc = acc + parts[0]
    ulb = ul_ref[ulb_off:ulb_off + nb * 2 * base, 0:nb * base]
    Z = dot(ulb, acc)
    for co in range(8):
        O = dot(Z[:, co * base:(co + 1) * base], urb_ref[...])
        for b in range(nb):
            o_ref[b, co] = O[b * 2 * base:(b + 1) * 2 * base, :]


def kernel(x0, x1, x2, x3, x4, w_0_0, b_0_0, w_0_1, b_0_1, w_0_2, b_0_2,
           w_0_3, b_0_3, w_0_4, b_0_4, w_1_0, b_1_0, w_1_1, b_1_1, w_1_2,
           b_1_2, w_1_3, b_1_3, w_2_0, b_2_0, w_2_1, b_2_1, w_2_2, b_2_2,
           w_3_0, b_3_0, w_3_1, b_3_1, w_4_0, b_4_0, conv_w, conv_b):
    ws = [[(w_0_0, b_0_0), (w_0_1, b_0_1), (w_0_2, b_0_2), (w_0_3, b_0_3),
           (w_0_4, b_0_4)],
          [(w_1_0, b_1_0), (w_1_1, b_1_1), (w_1_2, b_1_2), (w_1_3, b_1_3)],
          [(w_2_0, b_2_0), (w_2_1, b_2_1), (w_2_2, b_2_2)],
          [(w_3_0, b_3_0), (w_3_1, b_3_1)],
          [(w_4_0, b_4_0)]]
    xs = [x0, x1, x2, x3, x4]
    n = x0.shape[0]
    nb = 2 if n % 2 == 0 else 1
    base = x4.shape[-1]
    sizes = [x.shape[-1] for x in xs]
    eye_b = np.eye(nb, dtype=np.float32)

    ul_blocks, ul_offs = [], {}
    row = 0
    need_ul = set()
    for i in range(5):
        h = sizes[i]
        for j in range(1, 5 - i):
            need_ul.add(("b" if j == 3 else "n", h))
            h *= 2
    ul_w = nb * base
    for key in sorted(need_ul):
        m = np.kron(eye_b, _up_np(key[0], key[1]))
        ul_blocks.append(np.pad(m, ((0, 0), (0, ul_w - m.shape[1]))))
        ul_offs[key] = row
        row += m.shape[0]
    ulb_np = _up_np("b", base)
    ulb_off = row
    m = np.kron(eye_b, ulb_np)
    ul_blocks.append(np.pad(m, ((0, 0), (0, ul_w - m.shape[1]))))
    ulpack = jnp.asarray(np.concatenate(ul_blocks, 0))
    urbil = jnp.asarray(np.ascontiguousarray(ulb_np.T))

    c32_rows, cf_rows, bias_rows = [], [], []
    c32_off = cf_off = 0
    plan = []
    for i in range(5):
        h = sizes[i]
        steps = []
        for j in range(5 - i):
            w3, b = ws[i][j]
            final = j == 4 - i
            if j == 0:
                ul = None
                h2 = h
                bands = _band(w3, h2)
            else:
                mode = "b" if j == 3 else "n"
                ul = (ul_offs[(mode, h)], h)
                h2 = 2 * h
                rblk = jnp.asarray(np.kron(np.eye(4, dtype=np.float32),
                                           _up_np(mode, h).T))
                bands = [jnp.dot(rblk, m) for m in _band(w3, h2)]
            rows, cols = bands[0].shape
            if final:
                off = cf_off
                cf_rows += [jnp.pad(m, ((0, 0), (0, 1024 - cols)))
                            for m in bands]
                cf_off += 3 * rows
            else:
                off = c32_off
                c32_rows += [jnp.pad(m, ((0, 0), (0, 2 * base - cols)))
                             for m in bands]
                c32_off += 3 * rows
            brow = jnp.pad(jnp.repeat(b, h2), (0, 1024 - b.shape[0] * h2))
            steps.append({"ul": ul, "c": (off, rows, cols), "final": final,
                          "hout": h2, "b": len(bias_rows)})
            bias_rows.append(brow)
            h = h2
        plan.append(steps)

    bcat = [sum(jnp.kron(conv_w[ky, kx],
                         jnp.asarray(np.eye(base, k=1 - kx,
                                            dtype=np.float32)))
                for kx in range(3)) for ky in range(3)]
    bcat = jnp.concatenate(bcat, 0).astype(jnp.bfloat16)
    catb = len(bias_rows)
    bias_rows.append(jnp.pad(jnp.repeat(conv_b, base), (0, 1024 - 8 * base)))

    c32 = jnp.concatenate(c32_rows, 0)
    cf = jnp.concatenate(cf_rows, 0).astype(jnp.bfloat16)
    biasp = jnp.stack(bias_rows, 0)

    kfn = functools.partial(_fb_kernel, plan=plan, catb=catb,
                            ulb_off=ulb_off, base=base, nb=nb)
    mats = [ulpack, c32, cf, bcat, biasp, urbil]

    def whole(a):
        return pl.BlockSpec(a.shape, lambda bi: (0,) * a.ndim)

    return pl.pallas_call(
        kfn,
        out_shape=jax.ShapeDtypeStruct((n, 8, 2 * base, 2 * base),
                                       jnp.float32),
        grid=(n // nb,),
        in_specs=([pl.BlockSpec((nb, 4, s, s), lambda bi: (bi, 0, 0, 0))
                   for s in sizes] + [whole(a) for a in mats]),
        out_specs=pl.BlockSpec((nb, 8, 2 * base, 2 * base),
                               lambda bi: (bi, 0, 0, 0)),
        compiler_params=pltpu.CompilerParams(
            dimension_semantics=("parallel",),
            vmem_limit_bytes=60 * 1024 * 1024),
    )(*xs, *mats)
